# baseline (speedup 1.0000x reference)
import jax
import jax.numpy as jnp
from jax import lax
from jax.experimental import pallas as pl
from jax.experimental.pallas import tpu as pltpu
from jax.experimental.pallas import tpu_sc as plsc

_B = 1024
_N = 100000
_D = 10
_K = 16
_CW = 128
_NC = 782
_NPAD = _NC * _CW
_NCP = 896
_BT = 16
_NW = 32
_QPW = _B // _NW
_BIGI = 1 << 29


def _tc_body(x_ref, xt_ref, gsq_ref, w_ref, b_ref,
             lg_ref, cv_ref, ci_ref, d2_ref):
    q = jnp.dot(x_ref[...], w_ref[...],
                preferred_element_type=jnp.float32) + b_ref[...]
    qsq = jnp.sum(q * q, axis=1, keepdims=True)
    cross = jnp.dot(q, xt_ref[...], preferred_element_type=jnp.float32)
    d2 = qsq - 2.0 * cross + gsq_ref[...]
    d2_ref[...] = d2
    d2r = d2.reshape(_BT, _NC, _CW)
    cv = jnp.min(d2r, axis=2)
    gidx = (lax.broadcasted_iota(jnp.int32, (_BT, _NC, _CW), 1) * _CW
            + lax.broadcasted_iota(jnp.int32, (_BT, _NC, _CW), 2))
    ci = jnp.min(jnp.where(d2r == cv[:, :, None], gidx, _NPAD), axis=2)
    pad = _NCP - _NC
    cv_ref[...] = jnp.concatenate(
        [cv, jnp.full((_BT, pad), jnp.inf, jnp.float32)], axis=1)
    ci_ref[...] = jnp.concatenate(
        [ci, jnp.full((_BT, pad), _BIGI, jnp.int32)], axis=1)
    mx = jnp.max(jnp.abs(q), axis=1, keepdims=True)
    lg_ref[...] = jnp.concatenate(
        [q, mx, jnp.zeros((_BT, 5), jnp.float32)], axis=1)


def _sc_body(cmv, cmi, lg, d2t, ysg2, adv,
             cvb, cib, qvb, vbuf, lbuf, advb, sem1, sem2):
    wid = lax.axis_index("s") * 2 + lax.axis_index("c")
    base = wid * _QPW
    lane = lax.iota(jnp.int32, 16)
    inf16 = jnp.full((16,), jnp.inf, jnp.float32)
    big16 = jnp.full((16,), _BIGI, jnp.int32)

    def merge16(rv, ri, kv, iv):
        ks, js = plsc.sort_key_val(kv, iv)
        kb = lax.rev(ks, (0,))
        ib = lax.rev(js, (0,))
        take = (rv < kb) | ((rv == kb) & (ri < ib))
        mv = jnp.where(take, rv, kb)
        mi = jnp.where(take, ri, ib)
        rv2, ri2 = plsc.sort_key_val(mv, mi)
        return rv2, ri2, jnp.max(rv2, axis=0)

    def q_body(qi, carry):
        acc0, acc1 = carry
        qq = base + qi
        pltpu.sync_copy(cmv.at[qq], cvb)
        pltpu.sync_copy(cmi.at[qq], cib)
        pltpu.sync_copy(lg.at[pl.ds(qq * 16, 16)], qvb)

        def p1(i, carry):
            rv, ri, th = carry
            kv = cvb[pl.ds(i * 16, 16)]
            iv = cib[pl.ds(i * 16, 16)]
            mn = jnp.min(kv, axis=0)
            return lax.cond(
                mn <= th,
                lambda op: merge16(*op),
                lambda op: (op[0], op[1], th),
                (rv, ri, kv, iv))
        rv, ri, _th = lax.fori_loop(0, _NCP // 16, p1,
                                    (inf16, big16, jnp.float32(jnp.inf)))
        cids = lax.shift_right_logical(ri, 7)

        rbase = qq * _NPAD
        cps = []
        for c in range(_K):
            coff = pl.multiple_of(cids[c] * _CW, 8)
            cps.append(pltpu.async_copy(
                d2t.at[pl.ds(rbase + coff, _CW)],
                vbuf.at[pl.ds(c * _CW, _CW)], sem1))
            cps.append(pltpu.async_copy(
                ysg2.at[pl.ds(coff, _CW)],
                lbuf.at[pl.ds(c * _CW, _CW)], sem2))
        for cp in cps:
            cp.wait()

        qv = qvb[...]
        mxq = qv[_D]

        def p2(c, carry):
            rv2, rp2, th2 = carry
            cvec = jnp.take_along_axis(cids, jnp.full((16,), c, jnp.int32),
                                       axis=0)
            for j in range(_CW // 16):
                off = j * 16
                val = vbuf[pl.ds(c * _CW + off, 16)]
                lab = lbuf[pl.ds(c * _CW + off, 16)]
                labbit = jnp.where(lab > 0.0, 1, 0).astype(jnp.int32)
                pk = ((cvec * _CW + off + lane) << 1) | labbit
                mn = jnp.min(val, axis=0)
                rv2, rp2, th2 = lax.cond(
                    mn <= th2,
                    lambda op: merge16(*op),
                    lambda op, th2=th2: (op[0], op[1], th2),
                    (rv2, rp2, val, pk))
            return rv2, rp2, th2
        rv2, rp2, _th2 = lax.fori_loop(
            0, _K, p2, (inf16, big16, jnp.float32(jnp.inf)))

        sgn = (2 * (rp2 & 1) - 1).astype(jnp.float32)
        votes = jnp.sum(sgn, axis=0)
        a = jnp.sign(votes) * 2.0 * mxq
        acc0 = jnp.where(lane == qi, a, acc0)
        acc1 = jnp.where(lane == qi - 16, a, acc1)
        return acc0, acc1

    zero16 = jnp.zeros((16,), jnp.float32)
    acc0, acc1 = lax.fori_loop(0, _QPW, q_body, (zero16, zero16))
    advb[pl.ds(0, 16)] = acc0
    advb[pl.ds(16, 16)] = acc1
    pltpu.sync_copy(advb, adv.at[pl.ds(base, _QPW)])


@jax.jit
def kernel(x, X, Y, W, b):
    f32 = jnp.float32
    gsqr = jnp.sum(X * X, axis=1)
    gsq = jnp.concatenate([gsqr, jnp.full((_NPAD - _N,), jnp.inf, f32)])
    xt = jnp.pad(X.T, ((0, 0), (0, _NPAD - _N)))
    ysgn = jnp.pad(2.0 * Y.astype(f32) - 1.0, (0, _NPAD - _N))

    lg16, cmv, cmi, d2a = pl.pallas_call(
        _tc_body,
        grid=(_B // _BT,),
        in_specs=[
            pl.BlockSpec((_BT, 512), lambda i: (i, 0)),
            pl.BlockSpec((_D, _NPAD), lambda i: (0, 0)),
            pl.BlockSpec((1, _NPAD), lambda i: (0, 0)),
            pl.BlockSpec((512, _D), lambda i: (0, 0)),
            pl.BlockSpec((1, _D), lambda i: (0, 0)),
        ],
        out_specs=[
            pl.BlockSpec((_BT, 16), lambda i: (i, 0)),
            pl.BlockSpec((_BT, _NCP), lambda i: (i, 0)),
            pl.BlockSpec((_BT, _NCP), lambda i: (i, 0)),
            pl.BlockSpec((_BT, _NPAD), lambda i: (i, 0)),
        ],
        out_shape=[
            jax.ShapeDtypeStruct((_B, 16), f32),
            jax.ShapeDtypeStruct((_B, _NCP), f32),
            jax.ShapeDtypeStruct((_B, _NCP), jnp.int32),
            jax.ShapeDtypeStruct((_B, _NPAD), f32),
        ],
    )(x, xt, gsq.reshape(1, _NPAD), W, b.reshape(1, _D))

    sc_fn = pl.kernel(
        _sc_body,
        out_type=jax.ShapeDtypeStruct((_B,), f32),
        mesh=plsc.VectorSubcoreMesh(core_axis_name="c", subcore_axis_name="s"),
        compiler_params=pltpu.CompilerParams(needs_layout_passes=False),
        scratch_types=[
            pltpu.VMEM((_NCP,), f32),
            pltpu.VMEM((_NCP,), jnp.int32),
            pltpu.VMEM((16,), f32),
            pltpu.VMEM((_K * _CW,), f32),
            pltpu.VMEM((_K * _CW,), f32),
            pltpu.VMEM((_QPW,), f32),
            pltpu.SemaphoreType.DMA,
            pltpu.SemaphoreType.DMA,
        ],
    )
    adv = sc_fn(cmv, cmi, lg16.reshape(-1), d2a.reshape(-1), ysgn)
    return jnp.concatenate([lg16[:, :_D], adv[:, None]], axis=1)


# speedup vs baseline: 1.3377x; 1.3377x over previous
import jax
import jax.numpy as jnp
from jax import lax
from jax.experimental import pallas as pl
from jax.experimental.pallas import tpu as pltpu
from jax.experimental.pallas import tpu_sc as plsc

_B = 1024
_N = 100000
_D = 10
_K = 16
_CW = 128
_NC = 782
_NPAD = _NC * _CW
_NCP = 896
_BT = 16
_NW = 32
_QPW = _B // _NW
_BIGI = 1 << 29


def _tc_body(x_ref, xt_ref, gsq_ref, w_ref, b_ref,
             lg_ref, cv_ref, ci_ref, d2_ref):
    q = jnp.dot(x_ref[...], w_ref[...],
                preferred_element_type=jnp.float32) + b_ref[...]
    qsq = jnp.sum(q * q, axis=1, keepdims=True)
    cross = jnp.dot(q, xt_ref[...], preferred_element_type=jnp.float32)
    d2 = qsq - 2.0 * cross + gsq_ref[...]
    d2_ref[...] = d2
    d2r = d2.reshape(_BT, _NC, _CW)
    cv = jnp.min(d2r, axis=2)
    gidx = (lax.broadcasted_iota(jnp.int32, (_BT, _NC, _CW), 1) * _CW
            + lax.broadcasted_iota(jnp.int32, (_BT, _NC, _CW), 2))
    ci = jnp.min(jnp.where(d2r == cv[:, :, None], gidx, _NPAD), axis=2)
    pad = _NCP - _NC
    cv_ref[...] = jnp.concatenate(
        [cv, jnp.full((_BT, pad), jnp.inf, jnp.float32)], axis=1)
    ci_ref[...] = jnp.concatenate(
        [ci, jnp.full((_BT, pad), _BIGI, jnp.int32)], axis=1)
    mx = jnp.max(jnp.abs(q), axis=1, keepdims=True)
    lg_ref[...] = jnp.concatenate(
        [q, mx, jnp.zeros((_BT, 5), jnp.float32)], axis=1)


def _sc_body(cmv, cmi, lg, d2t, ysg2, adv,
             cvb, cib, qvb, vbuf, lbuf, advb, sem1, sem2):
    wid = lax.axis_index("s") * 2 + lax.axis_index("c")
    base = wid * _QPW
    lane = lax.iota(jnp.int32, 16)
    inf16 = jnp.full((16,), jnp.inf, jnp.float32)
    big16 = jnp.full((16,), _BIGI, jnp.int32)

    def merge16(rv, ri, kv, iv):
        ks, js = plsc.sort_key_val(kv, iv)
        kb = lax.rev(ks, (0,))
        ib = lax.rev(js, (0,))
        take = (rv < kb) | ((rv == kb) & (ri < ib))
        mv = jnp.where(take, rv, kb)
        mi = jnp.where(take, ri, ib)
        rv2, ri2 = plsc.sort_key_val(mv, mi)
        return rv2, ri2, jnp.max(rv2, axis=0)

    def q_body(qi, carry):
        acc0, acc1 = carry
        qq = base + qi
        pltpu.sync_copy(cmv.at[qq], cvb)
        pltpu.sync_copy(cmi.at[qq], cib)
        pltpu.sync_copy(lg.at[qq], qvb)

        def p1(i, carry):
            rv, ri, th = carry
            kv = cvb[pl.ds(i * 16, 16)]
            iv = cib[pl.ds(i * 16, 16)]
            mn = jnp.min(kv, axis=0)
            return lax.cond(
                mn <= th,
                lambda op: merge16(*op),
                lambda op: (op[0], op[1], th),
                (rv, ri, kv, iv))
        rv, ri, _th = lax.fori_loop(0, _NCP // 16, p1,
                                    (inf16, big16, jnp.float32(jnp.inf)))
        cids = lax.shift_right_logical(ri, 7)

        cps = []
        for c in range(_K):
            coff = pl.multiple_of(cids[c] * _CW, 8)
            cps.append(pltpu.async_copy(
                d2t.at[qq, pl.ds(coff, _CW)],
                vbuf.at[pl.ds(c * _CW, _CW)], sem1))
            cps.append(pltpu.async_copy(
                ysg2.at[pl.ds(coff, _CW)],
                lbuf.at[pl.ds(c * _CW, _CW)], sem2))
        for cp in cps:
            cp.wait()

        qv = qvb[...]
        mxq = qv[_D]

        def p2(c, carry):
            rv2, rp2, th2 = carry
            cvec = jnp.take_along_axis(cids, jnp.full((16,), c, jnp.int32),
                                       axis=0)
            for j in range(_CW // 16):
                off = j * 16
                val = vbuf[pl.ds(c * _CW + off, 16)]
                lab = lbuf[pl.ds(c * _CW + off, 16)]
                labbit = jnp.where(lab > 0.0, 1, 0).astype(jnp.int32)
                pk = ((cvec * _CW + off + lane) << 1) | labbit
                mn = jnp.min(val, axis=0)
                rv2, rp2, th2 = lax.cond(
                    mn <= th2,
                    lambda op: merge16(*op),
                    lambda op, th2=th2: (op[0], op[1], th2),
                    (rv2, rp2, val, pk))
            return rv2, rp2, th2
        rv2, rp2, _th2 = lax.fori_loop(
            0, _K, p2, (inf16, big16, jnp.float32(jnp.inf)))

        sgn = (2 * (rp2 & 1) - 1).astype(jnp.float32)
        votes = jnp.sum(sgn, axis=0)
        a = jnp.sign(votes) * 2.0 * mxq
        acc0 = jnp.where(lane == qi, a, acc0)
        acc1 = jnp.where(lane == qi - 16, a, acc1)
        return acc0, acc1

    zero16 = jnp.zeros((16,), jnp.float32)
    acc0, acc1 = lax.fori_loop(0, _QPW, q_body, (zero16, zero16))
    advb[pl.ds(0, 16)] = acc0
    advb[pl.ds(16, 16)] = acc1
    pltpu.sync_copy(advb, adv.at[pl.ds(base, _QPW)])


@jax.jit
def kernel(x, X, Y, W, b):
    f32 = jnp.float32
    gsqr = jnp.sum(X * X, axis=1)
    gsq = jnp.concatenate([gsqr, jnp.full((_NPAD - _N,), jnp.inf, f32)])
    xt = jnp.pad(X.T, ((0, 0), (0, _NPAD - _N)))
    ysgn = jnp.pad(2.0 * Y.astype(f32) - 1.0, (0, _NPAD - _N))

    lg16, cmv, cmi, d2a = pl.pallas_call(
        _tc_body,
        grid=(_B // _BT,),
        in_specs=[
            pl.BlockSpec((_BT, 512), lambda i: (i, 0)),
            pl.BlockSpec((_D, _NPAD), lambda i: (0, 0)),
            pl.BlockSpec((1, _NPAD), lambda i: (0, 0)),
            pl.BlockSpec((512, _D), lambda i: (0, 0)),
            pl.BlockSpec((1, _D), lambda i: (0, 0)),
        ],
        out_specs=[
            pl.BlockSpec((_BT, 16), lambda i: (i, 0)),
            pl.BlockSpec((_BT, _NCP), lambda i: (i, 0)),
            pl.BlockSpec((_BT, _NCP), lambda i: (i, 0)),
            pl.BlockSpec((_BT, _NPAD), lambda i: (i, 0)),
        ],
        out_shape=[
            jax.ShapeDtypeStruct((_B, 16), f32),
            jax.ShapeDtypeStruct((_B, _NCP), f32),
            jax.ShapeDtypeStruct((_B, _NCP), jnp.int32),
            jax.ShapeDtypeStruct((_B, _NPAD), f32),
        ],
    )(x, xt, gsq.reshape(1, _NPAD), W, b.reshape(1, _D))

    sc_fn = pl.kernel(
        _sc_body,
        out_type=jax.ShapeDtypeStruct((_B,), f32),
        mesh=plsc.VectorSubcoreMesh(core_axis_name="c", subcore_axis_name="s"),
        compiler_params=pltpu.CompilerParams(needs_layout_passes=False),
        scratch_types=[
            pltpu.VMEM((_NCP,), f32),
            pltpu.VMEM((_NCP,), jnp.int32),
            pltpu.VMEM((16,), f32),
            pltpu.VMEM((_K * _CW,), f32),
            pltpu.VMEM((_K * _CW,), f32),
            pltpu.VMEM((_QPW,), f32),
            pltpu.SemaphoreType.DMA,
            pltpu.SemaphoreType.DMA,
        ],
    )
    adv = sc_fn(cmv, cmi, lg16, d2a, ysgn)
    return jnp.concatenate([lg16[:, :_D], adv[:, None]], axis=1)


# speedup vs baseline: 1.9633x; 1.4676x over previous
import jax
import jax.numpy as jnp
from jax import lax
from jax.experimental import pallas as pl
from jax.experimental.pallas import tpu as pltpu
from jax.experimental.pallas import tpu_sc as plsc

_B = 1024
_N = 100000
_D = 10
_K = 16
_CW = 128
_NC = 782
_NPAD = _NC * _CW
_NCP = 896
_BT = 16
_NW = 32
_QPW = _B // _NW
_BIGI = 1 << 29


def _tc_body(x_ref, xt_ref, gsq_ref, w_ref, b_ref,
             lg_ref, cv_ref, d2_ref):
    q = jnp.dot(x_ref[...], w_ref[...],
                preferred_element_type=jnp.float32) + b_ref[...]
    qsq = jnp.sum(q * q, axis=1, keepdims=True)
    cross = jnp.dot(q, xt_ref[...], preferred_element_type=jnp.float32)
    d2 = qsq - 2.0 * cross + gsq_ref[...]
    d2_ref[...] = d2
    d2r = d2.reshape(_BT, _NC, _CW)
    cv = jnp.min(d2r, axis=2)
    pad = _NCP - _NC
    cv_ref[...] = jnp.concatenate(
        [cv, jnp.full((_BT, pad), jnp.inf, jnp.float32)], axis=1)
    mx = jnp.max(jnp.abs(q), axis=1, keepdims=True)
    lg_ref[...] = jnp.concatenate(
        [q, mx, jnp.zeros((_BT, 5), jnp.float32)], axis=1)


def _sc_body(cmv, lg, d2t, ysg2, adv,
             cvb, qvb, vbuf, ysv, advb, sem1):
    wid = lax.axis_index("s") * 2 + lax.axis_index("c")
    base = wid * _QPW
    pltpu.sync_copy(ysg2, ysv)
    lane = lax.iota(jnp.int32, 16)
    inf16 = jnp.full((16,), jnp.inf, jnp.float32)
    big16 = jnp.full((16,), _BIGI, jnp.int32)

    def merge16(rv, ri, kv, iv):
        ks, js = plsc.sort_key_val(kv, iv)
        kb = lax.rev(ks, (0,))
        ib = lax.rev(js, (0,))
        take = (rv < kb) | ((rv == kb) & (ri < ib))
        mv = jnp.where(take, rv, kb)
        mi = jnp.where(take, ri, ib)
        rv2, ri2 = plsc.sort_key_val(mv, mi)
        return rv2, ri2, jnp.max(rv2, axis=0)

    def q_body(qi, carry):
        acc0, acc1 = carry
        qq = base + qi
        pltpu.sync_copy(cmv.at[qq], cvb)
        pltpu.sync_copy(lg.at[qq], qvb)

        def p1(i, carry):
            rv, ri, th = carry
            kv = cvb[pl.ds(i * 16, 16)]
            iv = i * 16 + lane
            mn = jnp.min(kv, axis=0)
            return lax.cond(
                mn <= th,
                lambda op: merge16(*op),
                lambda op: (op[0], op[1], th),
                (rv, ri, kv, iv))
        rv, cids, _th = lax.fori_loop(0, _NCP // 16, p1,
                                      (inf16, big16, jnp.float32(jnp.inf)))

        cps = []
        for c in range(_K):
            coff = pl.multiple_of(cids[c] * _CW, 8)
            cps.append(pltpu.async_copy(
                d2t.at[qq, pl.ds(coff, _CW)],
                vbuf.at[pl.ds(c * _CW, _CW)], sem1))
        for cp in cps:
            cp.wait()

        qv = qvb[...]
        mxq = qv[_D]

        def p2(c, carry):
            rv2, rp2, th2 = carry
            cvec = jnp.take_along_axis(cids, jnp.full((16,), c, jnp.int32),
                                       axis=0)
            for j in range(_CW // 16):
                off = j * 16
                val = vbuf[pl.ds(c * _CW + off, 16)]
                pk = cvec * _CW + off + lane
                mn = jnp.min(val, axis=0)
                rv2, rp2, th2 = lax.cond(
                    mn <= th2,
                    lambda op: merge16(*op),
                    lambda op, th2=th2: (op[0], op[1], th2),
                    (rv2, rp2, val, pk))
            return rv2, rp2, th2
        rv2, rp2, _th2 = lax.fori_loop(
            0, _K, p2, (inf16, big16, jnp.float32(jnp.inf)))

        labs = plsc.load_gather(ysv, [rp2])
        sgn = jnp.where(labs > 0.0, 1.0, -1.0).astype(jnp.float32)
        votes = jnp.sum(sgn, axis=0)
        a = jnp.sign(votes) * 2.0 * mxq
        acc0 = jnp.where(lane == qi, a, acc0)
        acc1 = jnp.where(lane == qi - 16, a, acc1)
        return acc0, acc1

    zero16 = jnp.zeros((16,), jnp.float32)
    acc0, acc1 = lax.fori_loop(0, _QPW, q_body, (zero16, zero16))
    advb[pl.ds(0, 16)] = acc0
    advb[pl.ds(16, 16)] = acc1
    pltpu.sync_copy(advb, adv.at[pl.ds(base, _QPW)])


@jax.jit
def kernel(x, X, Y, W, b):
    f32 = jnp.float32
    gsqr = jnp.sum(X * X, axis=1)
    gsq = jnp.concatenate([gsqr, jnp.full((_NPAD - _N,), jnp.inf, f32)])
    xt = jnp.pad(X.T, ((0, 0), (0, _NPAD - _N)))
    ysgn = jnp.pad(2.0 * Y.astype(f32) - 1.0, (0, _NPAD - _N))

    lg16, cmv, d2a = pl.pallas_call(
        _tc_body,
        grid=(_B // _BT,),
        in_specs=[
            pl.BlockSpec((_BT, 512), lambda i: (i, 0)),
            pl.BlockSpec((_D, _NPAD), lambda i: (0, 0)),
            pl.BlockSpec((1, _NPAD), lambda i: (0, 0)),
            pl.BlockSpec((512, _D), lambda i: (0, 0)),
            pl.BlockSpec((1, _D), lambda i: (0, 0)),
        ],
        out_specs=[
            pl.BlockSpec((_BT, 16), lambda i: (i, 0)),
            pl.BlockSpec((_BT, _NCP), lambda i: (i, 0)),
            pl.BlockSpec((_BT, _NPAD), lambda i: (i, 0)),
        ],
        out_shape=[
            jax.ShapeDtypeStruct((_B, 16), f32),
            jax.ShapeDtypeStruct((_B, _NCP), f32),
            jax.ShapeDtypeStruct((_B, _NPAD), f32),
        ],
    )(x, xt, gsq.reshape(1, _NPAD), W, b.reshape(1, _D))

    sc_fn = pl.kernel(
        _sc_body,
        out_type=jax.ShapeDtypeStruct((_B,), f32),
        mesh=plsc.VectorSubcoreMesh(core_axis_name="c", subcore_axis_name="s"),
        compiler_params=pltpu.CompilerParams(needs_layout_passes=False),
        scratch_types=[
            pltpu.VMEM((_NCP,), f32),
            pltpu.VMEM((16,), f32),
            pltpu.VMEM((_K * _CW,), f32),
            pltpu.VMEM((_NPAD,), f32),
            pltpu.VMEM((_QPW,), f32),
            pltpu.SemaphoreType.DMA,
        ],
    )
    adv = sc_fn(cmv, lg16, d2a, ysgn)
    return jnp.concatenate([lg16[:, :_D], adv[:, None]], axis=1)
